# baseline (speedup 1.0000x reference)
import jax
import jax.numpy as jnp
from jax.experimental import pallas as pl
from jax.experimental.pallas import tpu as pltpu

MEM = 20000
FEAT = 3 * 32 * 32
NCLS = 100
PK = NCLS + 2
BATCH = 4096
COPY_ROWS = 256


def _copy_body(bx_ref, pk_ref, obx_ref, opk_ref):
    obx_ref[...] = bx_ref[...]
    opk_ref[...] = pk_ref[...]


def _scatter_body(idx_ref, x_ref, pk_ref, abx_ref, apk_ref, obx_ref, opk_ref):
    del idx_ref, abx_ref, apk_ref
    obx_ref[...] = x_ref[...]
    opk_ref[...] = pk_ref[...]


def kernel(x, y, logits, t, idx, bx, by, bt, blogits):
    xf = x.reshape(BATCH, FEAT)
    bxf = bx.reshape(MEM, FEAT)

    logits_bits = jax.lax.bitcast_convert_type(logits, jnp.int32)
    t_col = jnp.full((BATCH, 1), t, dtype=jnp.int32)
    pk_in = jnp.concatenate([logits_bits, y[:, None], t_col], axis=1)

    blogits_bits = jax.lax.bitcast_convert_type(blogits, jnp.int32)
    pk_buf = jnp.concatenate([blogits_bits, by[:, None], bt[:, None]], axis=1)

    cbx, cpk = pl.pallas_call(
        _copy_body,
        grid=(pl.cdiv(MEM, COPY_ROWS),),
        in_specs=[
            pl.BlockSpec((COPY_ROWS, FEAT), lambda i: (i, 0)),
            pl.BlockSpec((COPY_ROWS, PK), lambda i: (i, 0)),
        ],
        out_specs=[
            pl.BlockSpec((COPY_ROWS, FEAT), lambda i: (i, 0)),
            pl.BlockSpec((COPY_ROWS, PK), lambda i: (i, 0)),
        ],
        out_shape=[
            jax.ShapeDtypeStruct((MEM, FEAT), jnp.float32),
            jax.ShapeDtypeStruct((MEM, PK), jnp.int32),
        ],
    )(bxf, pk_buf)

    obx, opk = pl.pallas_call(
        _scatter_body,
        grid_spec=pltpu.PrefetchScalarGridSpec(
            num_scalar_prefetch=1,
            grid=(BATCH,),
            in_specs=[
                pl.BlockSpec((1, 1, FEAT), lambda i, idx_ref: (i, 0, 0)),
                pl.BlockSpec((1, 1, PK), lambda i, idx_ref: (i, 0, 0)),
                pl.BlockSpec(memory_space=pl.ANY),
                pl.BlockSpec(memory_space=pl.ANY),
            ],
            out_specs=[
                pl.BlockSpec((1, 1, FEAT), lambda i, idx_ref: (idx_ref[i], 0, 0)),
                pl.BlockSpec((1, 1, PK), lambda i, idx_ref: (idx_ref[i], 0, 0)),
            ],
        ),
        out_shape=[
            jax.ShapeDtypeStruct((MEM, 1, FEAT), jnp.float32),
            jax.ShapeDtypeStruct((MEM, 1, PK), jnp.int32),
        ],
        input_output_aliases={3: 0, 4: 1},
    )(idx, xf.reshape(BATCH, 1, FEAT), pk_in.reshape(BATCH, 1, PK),
      cbx.reshape(MEM, 1, FEAT), cpk.reshape(MEM, 1, PK))

    opk = opk.reshape(MEM, PK)
    bx_new = obx.reshape(MEM, 3, 32, 32)
    blogits_new = jax.lax.bitcast_convert_type(opk[:, :NCLS], jnp.float32)
    by_new = opk[:, NCLS]
    bt_new = opk[:, NCLS + 1]
    return (bx_new, by_new, bt_new, blogits_new)
